# baseline (speedup 1.0000x reference)
import functools

import jax
import jax.numpy as jnp
from jax import lax
from jax.experimental import pallas as pl
from jax.experimental.pallas import tpu as pltpu
from jax.experimental.pallas import tpu_sc as plsc

HIDDEN = 64
NUM_CORES = 2
NUM_SUBCORES = 16
NUM_WORKERS = NUM_CORES * NUM_SUBCORES
CHUNK = 400
NBUF = 4
LAG = 2


@functools.partial(jax.jit, static_argnames=("total",))
def _sc_gather(idx_flat, table, total):
    rows_per_worker = total // NUM_WORKERS
    num_chunks = rows_per_worker // CHUNK
    mesh = plsc.VectorSubcoreMesh(core_axis_name="c", subcore_axis_name="s")

    @functools.partial(
        pl.kernel,
        mesh=mesh,
        out_type=jax.ShapeDtypeStruct((total, HIDDEN), jnp.float32),
        compiler_params=pltpu.CompilerParams(use_tc_tiling_on_sc=False),
        scratch_types=[
            pltpu.VMEM((rows_per_worker,), jnp.int32),
            pltpu.VMEM((NBUF, CHUNK, HIDDEN), jnp.float32),
            pltpu.SemaphoreType.DMA((NBUF,)),
            pltpu.SemaphoreType.DMA((NBUF,)),
        ],
    )
    def k(idx_hbm, table_hbm, out_hbm, idx_v, rows_v, gsem, osem):
        wid = lax.axis_index("s") * NUM_CORES + lax.axis_index("c")
        base = wid * rows_per_worker
        pltpu.sync_copy(idx_hbm.at[pl.ds(base, rows_per_worker)], idx_v)

        def fire_gather(c):
            b = c % NBUF
            return pltpu.async_copy(
                table_hbm.at[idx_v.at[pl.ds(c * CHUNK, CHUNK)]],
                rows_v.at[b], gsem.at[b])

        def fire_out(c):
            b = c % NBUF
            return pltpu.async_copy(
                rows_v.at[b], out_hbm.at[pl.ds(base + c * CHUNK, CHUNK)],
                osem.at[b])

        gathers = {c: fire_gather(c) for c in range(min(NBUF, num_chunks))}
        outs = {}
        for c in range(num_chunks):
            if c >= LAG:
                outs[c - LAG].wait()
            nf = c + NBUF - LAG
            if NBUF <= nf < num_chunks:
                gathers[nf] = fire_gather(nf)
            gathers[c].wait()
            outs[c] = fire_out(c)
        for c in range(max(0, num_chunks - LAG), num_chunks):
            outs[c].wait()

    return k(idx_flat, table)


VBLK = 32768


def _tc_to_rowmajor(table_t):
    _, v = table_t.shape
    grid = (v + VBLK - 1) // VBLK
    hb = VBLK // 2

    def body(x_ref, o_ref):
        o_ref[:, 0:64] = x_ref[:, :hb].T
        o_ref[:, 64:128] = x_ref[:, hb:].T

    return pl.pallas_call(
        body,
        grid=(grid,),
        in_specs=[pl.BlockSpec((64, VBLK), lambda i: (0, i))],
        out_specs=pl.BlockSpec((hb, 128), lambda i: (i, 0)),
        out_shape=jax.ShapeDtypeStruct((grid * hb, 128), jnp.float32),
    )(table_t)


def kernel(input_, shared_weights):
    idx = input_.reshape(-1).astype(jnp.int32)
    v, h = shared_weights.shape
    hb = VBLK // 2
    r = idx % VBLK
    idx = (idx // VBLK) * VBLK + 2 * (r % hb) + r // hb
    t2 = _tc_to_rowmajor(shared_weights.T)
    t3 = t2.reshape(t2.shape[0] * 2, h)
    out = _sc_gather(idx, t3, idx.shape[0])
    return out.reshape(input_.shape + (HIDDEN,))


# speedup vs baseline: 1.2558x; 1.2558x over previous
import functools

import jax
import jax.numpy as jnp
from jax import lax
from jax.experimental import pallas as pl
from jax.experimental.pallas import tpu as pltpu
from jax.experimental.pallas import tpu_sc as plsc

HIDDEN = 64
NUM_CORES = 2
NUM_SUBCORES = 16
NUM_WORKERS = NUM_CORES * NUM_SUBCORES
CHUNK = 400
NBUF = 4
LAG = 2


@functools.partial(jax.jit, static_argnames=("nb", "ns"))
def _sc_gather(idx_flat, table, nb, ns):
    ns_pad = (ns + 7) // 8 * 8
    total = nb * ns
    rows_per_worker = total // NUM_WORKERS
    num_chunks = rows_per_worker // CHUNK
    b_per_chunk = CHUNK // ns
    mesh = plsc.VectorSubcoreMesh(core_axis_name="c", subcore_axis_name="s")

    @functools.partial(
        pl.kernel,
        mesh=mesh,
        out_type=jax.ShapeDtypeStruct((nb, ns_pad, 2 * HIDDEN), jnp.float32),
        compiler_params=pltpu.CompilerParams(use_tc_tiling_on_sc=False),
        scratch_types=[
            pltpu.VMEM((rows_per_worker,), jnp.int32),
            pltpu.VMEM((NBUF, CHUNK, HIDDEN), jnp.float32),
            pltpu.SemaphoreType.DMA((NBUF,)),
            pltpu.SemaphoreType.DMA((NBUF,)),
        ],
    )
    def k(idx_hbm, table_hbm, out_hbm, idx_v, rows_v, gsem, osem):
        wid = lax.axis_index("s") * NUM_CORES + lax.axis_index("c")
        base = wid * rows_per_worker
        pltpu.sync_copy(idx_hbm.at[pl.ds(base, rows_per_worker)], idx_v)

        def fire_gather(c):
            b = c % NBUF
            return pltpu.async_copy(
                table_hbm.at[idx_v.at[pl.ds(c * CHUNK, CHUNK)]],
                rows_v.at[b], gsem.at[b])

        def fire_out(c):
            b = c % NBUF
            b0 = (base + c * CHUNK) // ns
            return [
                pltpu.async_copy(
                    rows_v.at[b, pl.ds(g * ns, ns), :],
                    out_hbm.at[b0 + g, pl.ds(0, ns), pl.ds(0, HIDDEN)],
                    osem.at[b])
                for g in range(b_per_chunk)
            ]

        gathers = {c: fire_gather(c) for c in range(min(NBUF, num_chunks))}
        outs = {}
        for c in range(num_chunks):
            if c >= LAG:
                for cp in outs.pop(c - LAG):
                    cp.wait()
            nf = c + NBUF - LAG
            if NBUF <= nf < num_chunks:
                gathers[nf] = fire_gather(nf)
            gathers[c].wait()
            outs[c] = fire_out(c)
        for c in sorted(outs):
            for cp in outs[c]:
                cp.wait()

    return k(idx_flat, table)


VBLK = 32768


def _tc_to_rowmajor(table_t):
    _, v = table_t.shape
    grid = (v + VBLK - 1) // VBLK
    hb = VBLK // 2

    def body(x_ref, o_ref):
        o_ref[:, 0:64] = x_ref[:, :hb].T
        o_ref[:, 64:128] = x_ref[:, hb:].T

    return pl.pallas_call(
        body,
        grid=(grid,),
        in_specs=[pl.BlockSpec((64, VBLK), lambda i: (0, i))],
        out_specs=pl.BlockSpec((hb, 128), lambda i: (i, 0)),
        out_shape=jax.ShapeDtypeStruct((grid * hb, 128), jnp.float32),
    )(table_t)


def kernel(input_, shared_weights):
    idx = input_.reshape(-1).astype(jnp.int32)
    v, h = shared_weights.shape
    hb = VBLK // 2
    r = idx % VBLK
    idx = (idx // VBLK) * VBLK + 2 * (r % hb) + r // hb
    t2 = _tc_to_rowmajor(shared_weights.T)
    t3 = t2.reshape(t2.shape[0] * 2, h)
    nb, ns = input_.shape
    out = _sc_gather(idx, t3, nb, ns)
    return out[:, :ns, :HIDDEN]
